# Initial kernel scaffold; baseline (speedup 1.0000x reference)
import jax
import jax.numpy as jnp
from jax.experimental import pallas as pl


def kernel(x1, edge_index1, x2, edge_index2, W_emb1, b_emb1, W_emb2, b_emb2, W_g1_0, b_g1_0, W_g1_1, b_g1_1, W_g2_0, b_g2_0, W_g2_1, b_g2_1, W_f, b_f):
    raise NotImplementedError("write your pallas kernel here")


# speedup vs baseline: 17.4664x; 17.4664x over previous
import functools

import jax
import jax.numpy as jnp
from jax import lax
from jax.experimental import pallas as pl
from jax.experimental.pallas import tpu as pltpu
from jax.experimental.pallas import tpu_sc as plsc

_N = 10000
_E = 320000
_D = 128
_NC = 2
_NS = 16
_EPT = _E // _NS
_CH = 128
_NFULL = _EPT // _CH
_TAIL = _EPT - _NFULL * _CH
_RC = 80
_NRC = _N // _RC
_NP = 10240
_DC = 128


def _sc_mesh():
  return plsc.VectorSubcoreMesh(core_axis_name="c", subcore_axis_name="s")


def _zero_acc(zsrc_vmem, acc, sid):
  for k in range(8):
    c = sid + 16 * k

    @pl.when(c < _NRC)
    def _():
      pltpu.sync_copy(zsrc_vmem.at[pl.ds(0, _RC)], acc.at[pl.ds(c * _RC, _RC)])


def _write_out(acc, out_hbm, cid, sid):
  for k in range(8):
    c = sid + 16 * k

    @pl.when(c < _NRC)
    def _():
      pltpu.sync_copy(acc.at[pl.ds(c * _RC, _RC)],
                      out_hbm.at[cid, pl.ds(c * _RC, _RC)])


def _sc_degree(dst_l, ones128, zeros128):
  @functools.partial(
      pl.kernel,
      out_type=jax.ShapeDtypeStruct((2 * _NP,), jnp.float32),
      mesh=_sc_mesh(),
      scratch_types=[
          pltpu.VMEM_SHARED((_NP,), jnp.float32),
          pltpu.VMEM((_CH,), jnp.float32),
          pltpu.VMEM((_CH,), jnp.float32),
          pltpu.VMEM((_CH,), jnp.int32),
          pltpu.VMEM((_TAIL,), jnp.int32),
      ],
  )
  def k(dst_hbm, ones_hbm, zeros_hbm, out_hbm, dacc, obuf, zbuf, di, dit):
    cid = lax.axis_index("c")
    sid = lax.axis_index("s")
    pltpu.sync_copy(ones_hbm, obuf)
    pltpu.sync_copy(zeros_hbm, zbuf)
    for kk in range(_NP // _DC // _NS):
      pltpu.sync_copy(zbuf, dacc.at[pl.ds((sid + 16 * kk) * _DC, _DC)])
    plsc.subcore_barrier()
    e0 = cid * _E + sid * _EPT

    @pl.loop(0, _NFULL)
    def _(i):
      pltpu.sync_copy(dst_hbm.at[pl.ds(e0 + i * _CH, _CH)], di)
      pltpu.sync_copy(obuf, dacc.at[di], add=True)

    offt = e0 + _NFULL * _CH
    pltpu.sync_copy(dst_hbm.at[pl.ds(offt, _TAIL)], dit)
    pltpu.sync_copy(obuf.at[pl.ds(0, _TAIL)], dacc.at[dit], add=True)
    plsc.subcore_barrier()
    for kk in range(_NP // _DC // _NS):
      c = (sid + 16 * kk) * _DC
      pltpu.sync_copy(dacc.at[pl.ds(c, _DC)],
                      out_hbm.at[pl.ds(cid * _NP + c, _DC)])

  return k(dst_l, ones128, zeros128)


def _sc_scatter_rows(q_cat, src_g, dst_l, zeros128):
  @functools.partial(
      pl.kernel,
      out_type=jax.ShapeDtypeStruct((2, _N, _D), jnp.float32),
      mesh=_sc_mesh(),
      scratch_types=[
          pltpu.VMEM_SHARED((_N, _D), jnp.float32),
          pltpu.VMEM((_CH, _D), jnp.float32),
          pltpu.VMEM((_CH, _D), jnp.float32),
          pltpu.VMEM((_CH,), jnp.int32),
          pltpu.VMEM((_CH,), jnp.int32),
          pltpu.VMEM((_CH,), jnp.int32),
          pltpu.VMEM((_CH,), jnp.int32),
          pltpu.VMEM((_TAIL,), jnp.int32),
          pltpu.VMEM((_TAIL,), jnp.int32),
          pltpu.VMEM((_TAIL, _D), jnp.float32),
          pltpu.SemaphoreType.DMA,
          pltpu.SemaphoreType.DMA,
      ],
  )
  def k(q_hbm, src_hbm, dst_hbm, z_hbm, out_hbm, acc,
        rows_a, rows_b, si_a, si_b, di_a, di_b, sit, dit, rows_t,
        sem_a, sem_b):
    cid = lax.axis_index("c")
    sid = lax.axis_index("s")
    pltpu.sync_copy(z_hbm, rows_a)
    _zero_acc(rows_a, acc, sid)
    plsc.subcore_barrier()
    e0 = cid * _E + sid * _EPT

    rows = (rows_a, rows_b)
    si = (si_a, si_b)
    di = (di_a, di_b)
    sem = (sem_a, sem_b)

    pltpu.sync_copy(src_hbm.at[pl.ds(e0, _CH)], si_a)
    pltpu.sync_copy(dst_hbm.at[pl.ds(e0, _CH)], di_a)
    pltpu.async_copy(q_hbm.at[si_a], rows_a, sem_a)

    @pl.loop(0, _NFULL, step=2)
    def _(i):
      for b in range(2):
        cur, nxt = rows[b], rows[(b + 1) % 2]
        @pl.when(i + b + 1 < _NFULL)
        def _():
          off = e0 + (i + b + 1) * _CH
          pltpu.sync_copy(src_hbm.at[pl.ds(off, _CH)], si[(b + 1) % 2])
          pltpu.sync_copy(dst_hbm.at[pl.ds(off, _CH)], di[(b + 1) % 2])
          pltpu.async_copy(q_hbm.at[si[(b + 1) % 2]], nxt, sem[(b + 1) % 2])

        pltpu.make_async_copy(q_hbm.at[si[b]], cur, sem[b]).wait()
        pltpu.sync_copy(cur, acc.at[di[b]], add=True)

    offt = e0 + _NFULL * _CH
    pltpu.sync_copy(src_hbm.at[pl.ds(offt, _TAIL)], sit)
    pltpu.sync_copy(dst_hbm.at[pl.ds(offt, _TAIL)], dit)
    pltpu.async_copy(q_hbm.at[sit], rows_t, sem_a).wait()
    pltpu.sync_copy(rows_t, acc.at[dit], add=True)

    plsc.subcore_barrier()
    _write_out(acc, out_hbm, cid, sid)

  return k(q_cat, src_g, dst_l, zeros128)


_R = 2000
_NT = _N // _R


def _tc_embed_scale(x, w_emb, b_emb, w_g0, deg3):
  def body(x_ref, we_ref, be_ref, wg_ref, dg_ref, o_ref, r_ref):
    i = pl.program_id(1)
    h = jnp.dot(x_ref[0], we_ref[0], preferred_element_type=jnp.float32)
    h = h + be_ref[0]
    dg = dg_ref[0, pl.ds(i, 1), :][0]
    r = lax.rsqrt(jnp.maximum(dg, 1.0))
    rc = jnp.reshape(r, (_R, 1))
    rb = jnp.broadcast_to(rc, (_R, _D))
    r_ref[0] = rb
    o_ref[0] = jnp.dot(h * rb, wg_ref[0], preferred_element_type=jnp.float32)

  return pl.pallas_call(
      body,
      grid=(2, _NT),
      in_specs=[
          pl.BlockSpec((1, _R, _D), lambda b, i: (b, i, 0)),
          pl.BlockSpec((1, _D, _D), lambda b, i: (b, 0, 0)),
          pl.BlockSpec((1, 1, _D), lambda b, i: (b, 0, 0)),
          pl.BlockSpec((1, _D, _D), lambda b, i: (b, 0, 0)),
          pl.BlockSpec((1, _NT, _R), lambda b, i: (b, 0, 0)),
      ],
      out_specs=[
          pl.BlockSpec((1, _R, _D), lambda b, i: (b, i, 0)),
          pl.BlockSpec((1, _R, _D), lambda b, i: (b, i, 0)),
      ],
      out_shape=[
          jax.ShapeDtypeStruct((2, _N, _D), jnp.float32),
          jax.ShapeDtypeStruct((2, _N, _D), jnp.float32),
      ],
  )(x, w_emb, b_emb, w_g0, deg3)


def _tc_layer_mid(agg, rbig, b_prev, w_next):
  def body(a_ref, r_ref, bp_ref, wn_ref, o_ref):
    r = r_ref[0]
    h = jnp.maximum(a_ref[0] * r + bp_ref[0], 0.0)
    o_ref[0] = jnp.dot(h * r, wn_ref[0], preferred_element_type=jnp.float32)

  return pl.pallas_call(
      body,
      grid=(2, _NT),
      in_specs=[
          pl.BlockSpec((1, _R, _D), lambda b, i: (b, i, 0)),
          pl.BlockSpec((1, _R, _D), lambda b, i: (b, i, 0)),
          pl.BlockSpec((1, 1, _D), lambda b, i: (b, 0, 0)),
          pl.BlockSpec((1, _D, _D), lambda b, i: (b, 0, 0)),
      ],
      out_specs=pl.BlockSpec((1, _R, _D), lambda b, i: (b, i, 0)),
      out_shape=jax.ShapeDtypeStruct((2, _N, _D), jnp.float32),
  )(agg, rbig, b_prev, w_next)


def _tc_layer_max(agg, rbig, b_prev):
  def body(a_ref, r_ref, bp_ref, o_ref):
    i = pl.program_id(1)
    h = jnp.maximum(a_ref[0] * r_ref[0] + bp_ref[0], 0.0)
    pm = jnp.max(h, axis=0)[None, None, :]

    @pl.when(i == 0)
    def _():
      o_ref[...] = jnp.full((1, 1, _D), -jnp.inf, jnp.float32)

    o_ref[...] = jnp.maximum(o_ref[...], pm)

  return pl.pallas_call(
      body,
      grid=(2, _NT),
      in_specs=[
          pl.BlockSpec((1, _R, _D), lambda b, i: (b, i, 0)),
          pl.BlockSpec((1, _R, _D), lambda b, i: (b, i, 0)),
          pl.BlockSpec((1, 1, _D), lambda b, i: (b, 0, 0)),
      ],
      out_specs=pl.BlockSpec((1, 1, _D), lambda b, i: (b, 0, 0)),
      out_shape=jax.ShapeDtypeStruct((2, 1, _D), jnp.float32),
  )(agg, rbig, b_prev)


def _tc_head(m, w_f_pad, b_f_pad):
  def body(m_ref, w_ref, b_ref, o_ref):
    feats = jnp.concatenate([m_ref[0], m_ref[1]], axis=-1)
    logits = jnp.dot(feats, w_ref[...], preferred_element_type=jnp.float32)
    logits = logits + b_ref[...]
    act = jnp.where(logits >= 0.0, logits, 0.01 * logits)
    lane = lax.broadcasted_iota(jnp.int32, (1, _D), 1)
    masked = jnp.where(lane < 2, act, -jnp.inf)
    mx = jnp.max(masked, axis=-1, keepdims=True)
    e = jnp.where(lane < 2, jnp.exp(masked - mx), 0.0)
    o_ref[...] = e / jnp.sum(e, axis=-1, keepdims=True)

  return pl.pallas_call(
      body,
      out_shape=jax.ShapeDtypeStruct((1, _D), jnp.float32),
  )(m, w_f_pad, b_f_pad)


def kernel(x1, edge_index1, x2, edge_index2, W_emb1, b_emb1, W_emb2, b_emb2,
           W_g1_0, b_g1_0, W_g1_1, b_g1_1, W_g2_0, b_g2_0, W_g2_1, b_g2_1,
           W_f, b_f):
  src1 = edge_index1[0].astype(jnp.int32)
  dst1 = edge_index1[1].astype(jnp.int32)
  src2 = edge_index2[0].astype(jnp.int32)
  dst2 = edge_index2[1].astype(jnp.int32)
  src_g = jnp.concatenate([src1, src2 + _N])
  dst_l = jnp.concatenate([dst1, dst2])

  ones1 = jnp.ones((_CH,), jnp.float32)
  zeros1 = jnp.zeros((_CH,), jnp.float32)
  zeros128 = jnp.zeros((_CH, _D), jnp.float32)

  x = jnp.stack([x1, x2])
  w_emb = jnp.stack([W_emb1, W_emb2])
  b_emb = jnp.stack([b_emb1, b_emb2])[:, None, :]
  w_l0 = jnp.stack([W_g1_0, W_g2_0])
  b_l0 = jnp.stack([b_g1_0, b_g2_0])[:, None, :]
  w_l1 = jnp.stack([W_g1_1, W_g2_1])
  b_l1 = jnp.stack([b_g1_1, b_g2_1])[:, None, :]

  degf = _sc_degree(dst_l, ones1, zeros1)
  deg3 = degf.reshape(2, _NP)[:, :_N].reshape(2, _NT, _R)

  q0, rbig = _tc_embed_scale(x, w_emb, b_emb, w_l0, deg3)
  agg0 = _sc_scatter_rows(q0.reshape(2 * _N, _D), src_g, dst_l, zeros128)
  q1 = _tc_layer_mid(agg0, rbig, b_l0, w_l1)
  agg1 = _sc_scatter_rows(q1.reshape(2 * _N, _D), src_g, dst_l, zeros128)
  m = _tc_layer_max(agg1, rbig, b_l1)

  w_f_pad = jnp.zeros((2 * _D, _D), jnp.float32).at[:, :2].set(W_f)
  b_f_pad = jnp.zeros((1, _D), jnp.float32).at[:, :2].set(b_f[None, :])
  out = _tc_head(m, w_f_pad, b_f_pad)
  return out[:, :2]
